# baseline (speedup 1.0000x reference)
import functools

import jax
import jax.numpy as jnp
from jax import lax
from jax.experimental import pallas as pl
from jax.experimental.pallas import tpu as pltpu
from jax.experimental.pallas import tpu_sc as plsc

N_NODES = 100000
DIM = 128
NC, NS = 2, 16
NW = NC * NS
CHUNK = 400
NCHUNKS = N_NODES // CHUNK
MAXK = (NCHUNKS + NW - 1) // NW


def _make_sc_gather():
    mesh = plsc.VectorSubcoreMesh(core_axis_name="c", subcore_axis_name="s")

    @functools.partial(
        pl.kernel,
        out_type=jax.ShapeDtypeStruct((N_NODES, DIM), jnp.float32),
        mesh=mesh,
        scratch_types=[
            pltpu.VMEM((CHUNK,), jnp.int32),
            pltpu.VMEM((CHUNK,), jnp.int32),
            pltpu.VMEM((CHUNK, DIM), jnp.float32),
            pltpu.VMEM((CHUNK, DIM), jnp.float32),
            pltpu.SemaphoreType.DMA,
            pltpu.SemaphoreType.DMA,
        ],
    )
    def sc_gather(deg_hbm, table_hbm, out_hbm,
                  idx0, idx1, rows0, rows1, sem0, sem1):
        wid = lax.axis_index("s") * NC + lax.axis_index("c")
        nk = (NCHUNKS - wid + NW - 1) // NW
        idxs, rows, sems = (idx0, idx1), (rows0, rows1), (sem0, sem1)

        def start(k, b):
            base = (wid + k * NW) * CHUNK
            pltpu.sync_copy(deg_hbm.at[pl.ds(base, CHUNK)], idxs[b])
            pltpu.async_copy(table_hbm.at[idxs[b]], rows[b], sems[b])

        def drain_and_store(k, b):
            base = (wid + k * NW) * CHUNK
            pltpu.make_async_copy(table_hbm.at[idxs[b]], rows[b], sems[b]).wait()
            pltpu.sync_copy(rows[b], out_hbm.at[pl.ds(base, CHUNK)])

        for b in range(2):
            pl.when(b < nk)(lambda b=b: start(b, b))

        def outer(i, _):
            k0 = i * 2
            for b in range(2):
                k = k0 + b

                @pl.when(k < nk)
                def _(k=k, b=b):
                    drain_and_store(k, b)
                    pl.when(k + 2 < nk)(lambda: start(k + 2, b))

            return 0

        lax.fori_loop(0, (MAXK + 1) // 2, outer, 0)

    return sc_gather


_sc_gather = _make_sc_gather()


def kernel(degrees, table):
    return _sc_gather(degrees.astype(jnp.int32), table)
